# baseline (speedup 1.0000x reference)
import jax
import jax.numpy as jnp
import numpy as np
from jax.experimental import pallas as pl

_FEATURE_NUM = 128
_ACTIVE_LO = 4
_N_ACT = _FEATURE_NUM - _ACTIVE_LO
_MIN_MAX = {4: (0.0, 1000.0), 5: (-10.0, 10.0), 6: (0.0, 1.0), 7: (0.0, 255.0)}


def _col_consts():
    cmin = np.zeros((1, _FEATURE_NUM), np.float32)
    cmax = np.ones((1, _FEATURE_NUM), np.float32)
    for c, (lo, hi) in _MIN_MAX.items():
        cmin[0, c] = lo
        cmax[0, c] = hi
    inv = 1.0 / (cmax - cmin)
    return cmin, inv


def _tc_body(x_ref, cmin_ref, inv_ref, ad_ref, norm_ref):
    x = x_ref[...]
    n = (x - cmin_ref[...]) * inv_ref[...]
    n = jnp.clip(n, 0.0, 1.0)
    bad = (x == -1.0) | jnp.isnan(x)
    n = jnp.where(bad, 0.0, n)
    na = n[:, _ACTIVE_LO:]
    norm_ref[...] = na
    ad_ref[0] = na
    ad_ref[1] = na * na
    ad_ref[2] = jnp.sqrt(na)


@jax.jit
def kernel(features):
    rows = features.shape[0]
    block = 1024
    grid = rows // block
    cmin, inv = _col_consts()
    ad_planes, norm = pl.pallas_call(
        _tc_body,
        grid=(grid,),
        in_specs=[
            pl.BlockSpec((block, _FEATURE_NUM), lambda i: (i, 0)),
            pl.BlockSpec((1, _FEATURE_NUM), lambda i: (0, 0)),
            pl.BlockSpec((1, _FEATURE_NUM), lambda i: (0, 0)),
        ],
        out_specs=[
            pl.BlockSpec((3, block, _N_ACT), lambda i: (0, i, 0)),
            pl.BlockSpec((block, _N_ACT), lambda i: (i, 0)),
        ],
        out_shape=[
            jax.ShapeDtypeStruct((3, rows, _N_ACT), jnp.float32),
            jax.ShapeDtypeStruct((rows, _N_ACT), jnp.float32),
        ],
    )(features, jnp.asarray(cmin), jnp.asarray(inv))
    return jnp.transpose(ad_planes, (1, 2, 0)), norm


# speedup vs baseline: 1.2915x; 1.2915x over previous
import jax
import jax.numpy as jnp
import numpy as np
from jax.experimental import pallas as pl

_FEATURE_NUM = 128
_ACTIVE_LO = 4
_N_ACT = _FEATURE_NUM - _ACTIVE_LO
_MIN_MAX = {4: (0.0, 1000.0), 5: (-10.0, 10.0), 6: (0.0, 1.0), 7: (0.0, 255.0)}


def _col_consts():
    cmin = np.zeros((1, _FEATURE_NUM), np.float32)
    cmax = np.ones((1, _FEATURE_NUM), np.float32)
    for c, (lo, hi) in _MIN_MAX.items():
        cmin[0, c] = lo
        cmax[0, c] = hi
    inv = 1.0 / (cmax - cmin)
    return cmin, inv


def _tc_body(x_ref, cmin_ref, inv_ref, ad_ref, norm_ref):
    x = x_ref[...]
    n = (x - cmin_ref[...]) * inv_ref[...]
    n = jnp.clip(n, 0.0, 1.0)
    bad = (x == -1.0) | jnp.isnan(x)
    n = jnp.where(bad, 0.0, n)
    na = n[:, _ACTIVE_LO:]
    norm_ref[...] = na
    ad_ref[0] = na
    ad_ref[1] = na * na
    ad_ref[2] = jnp.sqrt(na)


@jax.jit
def kernel(features):
    rows = features.shape[0]
    block = 4096
    grid = rows // block
    cmin, inv = _col_consts()
    ad_planes, norm = pl.pallas_call(
        _tc_body,
        grid=(grid,),
        in_specs=[
            pl.BlockSpec((block, _FEATURE_NUM), lambda i: (i, 0)),
            pl.BlockSpec((1, _FEATURE_NUM), lambda i: (0, 0)),
            pl.BlockSpec((1, _FEATURE_NUM), lambda i: (0, 0)),
        ],
        out_specs=[
            pl.BlockSpec((3, block, _N_ACT), lambda i: (0, i, 0)),
            pl.BlockSpec((block, _N_ACT), lambda i: (i, 0)),
        ],
        out_shape=[
            jax.ShapeDtypeStruct((3, rows, _N_ACT), jnp.float32),
            jax.ShapeDtypeStruct((rows, _N_ACT), jnp.float32),
        ],
    )(features, jnp.asarray(cmin), jnp.asarray(inv))
    return jnp.transpose(ad_planes, (1, 2, 0)), norm
